# baseline (speedup 1.0000x reference)
import functools

import jax
import jax.numpy as jnp
from jax import lax
from jax.experimental import pallas as pl
from jax.experimental.pallas import tpu as pltpu
from jax.experimental.pallas import tpu_sc as plsc

K = 4
B = 8
L = 32
N = 64
SIZE = 128
KN = K * N
BL = B * L
CB = 32
NEG = float("-inf")
BIG = 2 ** 30


def _score_kernel(lh_ref, rh_ref, ls_ref, rs_ref, mat_ref, s_ref):
    mat = mat_ref[...]
    row_i = jax.lax.broadcasted_iota(jnp.int32, (KN, KN), 0)
    col_i = jax.lax.broadcasted_iota(jnp.int32, (KN, KN), 1)
    diag_idx = ((jax.lax.broadcasted_iota(jnp.int32, (KN, 2), 0) & (N - 1))
                + jax.lax.broadcasted_iota(jnp.int32, (KN, 2), 1) * N)
    dn_rowdot = (((1,), (1,)), ((), ()))

    mat_bf = mat.astype(jnp.bfloat16)
    for c in range(CB):
        lh_all = lh_ref[:, c].reshape(KN, SIZE)
        rh_all = rh_ref[:, c].reshape(KN, SIZE)
        lm_all = jnp.dot(lh_all.astype(jnp.bfloat16), mat_bf,
                         preferred_element_type=jnp.float32)
        d_all = jax.lax.dot_general(lm_all.astype(jnp.bfloat16),
                                    rh_all.astype(jnp.bfloat16), dn_rowdot,
                                    preferred_element_type=jnp.float32)
        s4 = jnp.concatenate(
            [jnp.take_along_axis(d_all[:, :2 * N], diag_idx, axis=1),
             jnp.take_along_axis(d_all[:, 2 * N:], diag_idx, axis=1)],
            axis=1)
        sraw = jnp.concatenate([s4[i1 * N:(i1 + 1) * N, :] for i1 in range(K)],
                               axis=1)
        ls4 = ls_ref[c].astype(jnp.bfloat16).astype(jnp.float32)
        rs4 = rs_ref[c].astype(jnp.bfloat16).astype(jnp.float32)
        ls16 = jnp.concatenate(
            [jnp.broadcast_to(ls4[:, i1:i1 + 1], (N, K)) for i1 in range(K)], axis=1)
        rs16 = jnp.concatenate([rs4] * K, axis=1)
        s_ref[c * N:(c + 1) * N, :] = (sraw + ls16) + rs16


CELLS_PER_W = BL // 32
CHUNKS = N


def _sc_select(s_hbm, lh_hbm, rh_hbm,
               lrows_hbm, rrows_hbm, outs_hbm, outg_hbm,
               s_v, lidx_v, ridx_v, lrows_v, rrows_v, outs_v, outg_v, sem):
    info = plsc.get_sparse_core_info()
    nc = info.num_cores
    wid = lax.axis_index("s") * nc + lax.axis_index("c")
    cell0 = wid * CELLS_PER_W

    pltpu.sync_copy(s_hbm.at[pl.ds(cell0 * N * (K * K), CELLS_PER_W * N * K * K)], s_v)

    lane = lax.iota(jnp.int32, 16)
    lvecs = [jnp.zeros((16,), jnp.int32) for _ in range(CELLS_PER_W // 4)]
    rvecs = [jnp.zeros((16,), jnp.int32) for _ in range(CELLS_PER_W // 4)]

    for c in range(CELLS_PER_W):
        sbase = c * (N * K * K)

        def scan_body(i, carry):
            v1, v2, v3, v4, g1, g2, g3, g4 = carry
            x = s_v[pl.ds(sbase + i * 16, 16)]
            fi = i * 16 + lane
            c1 = x > v1
            nv1 = jnp.where(c1, x, v1); r1 = jnp.where(c1, v1, x)
            ng1 = jnp.where(c1, fi, g1); h1 = jnp.where(c1, g1, fi)
            c2 = r1 > v2
            nv2 = jnp.where(c2, r1, v2); r2 = jnp.where(c2, v2, r1)
            ng2 = jnp.where(c2, h1, g2); h2 = jnp.where(c2, g2, h1)
            c3 = r2 > v3
            nv3 = jnp.where(c3, r2, v3); r3 = jnp.where(c3, v3, r2)
            ng3 = jnp.where(c3, h2, g3); h3 = jnp.where(c3, g3, h2)
            c4 = r3 > v4
            nv4 = jnp.where(c4, r3, v4)
            ng4 = jnp.where(c4, h3, g4)
            return nv1, nv2, nv3, nv4, ng1, ng2, ng3, ng4

        neg = jnp.full((16,), NEG)
        big = jnp.full((16,), BIG)
        v1, v2, v3, v4, g1, g2, g3, g4 = lax.fori_loop(
            0, CHUNKS, scan_body, (neg, neg, neg, neg, big, big, big, big))

        res_s = jnp.zeros((16,), jnp.float32)
        res_g = jnp.zeros((16,), jnp.int32)

        dnum = lax.GatherDimensionNumbers(
            offset_dims=(), collapsed_slice_dims=(0,), start_index_map=(0,))

        def perm(x, idx):
            return lax.gather(x, idx[:, None], dnum, (1,),
                              mode=lax.GatherScatterMode.PROMISE_IN_BOUNDS)

        def allmax(x):
            for sh in (1, 2, 4, 8):
                x = jnp.maximum(x, perm(x, lane ^ sh))
            return x

        def allmin(x):
            for sh in (1, 2, 4, 8):
                x = jnp.minimum(x, perm(x, lane ^ sh))
            return x

        for k in range(K):
            m = allmax(v1)
            cand = jnp.where(v1 == m, g1, BIG)
            g = allmin(cand)
            hit = cand == g
            v1 = jnp.where(hit, v2, v1); v2 = jnp.where(hit, v3, v2)
            v3 = jnp.where(hit, v4, v3); v4 = jnp.where(hit, NEG, v4)
            g1 = jnp.where(hit, g2, g1); g2 = jnp.where(hit, g3, g2)
            g3 = jnp.where(hit, g4, g3); g4 = jnp.where(hit, BIG, g4)
            n_ = g >> 4
            lk = (g >> 2) & 3
            rk = g & 3
            res_s = jnp.where(lane == k, m, res_s)
            res_g = jnp.where(lane == k, g, res_g)
            res_g = jnp.where(lane == 4 + k, n_, res_g)
            res_g = jnp.where(lane == 8 + k, lk, res_g)
            res_g = jnp.where(lane == 12 + k, rk, res_g)
            lrow = (lk * BL + (cell0 + c)) * N + n_
            rrow = (rk * BL + (cell0 + c)) * N + n_
            slot = (c % 4) * 4 + k
            lvecs[c // 4] = jnp.where(lane == slot, lrow, lvecs[c // 4])
            rvecs[c // 4] = jnp.where(lane == slot, rrow, rvecs[c // 4])

        outs_v[pl.ds(c * 16, 16)] = res_s
        outg_v[pl.ds(c * 16, 16)] = res_g

    for q in range(CELLS_PER_W // 4):
        lidx_v[pl.ds(q * 16, 16)] = lvecs[q]
        ridx_v[pl.ds(q * 16, 16)] = rvecs[q]

    pltpu.async_copy(lh_hbm.at[lidx_v], lrows_v, sem).wait()
    pltpu.async_copy(rh_hbm.at[ridx_v], rrows_v, sem).wait()

    rows0 = wid * (CELLS_PER_W * K)
    pltpu.sync_copy(lrows_v, lrows_hbm.at[pl.ds(rows0, CELLS_PER_W * K)])
    pltpu.sync_copy(rrows_v, rrows_hbm.at[pl.ds(rows0, CELLS_PER_W * K)])
    pltpu.sync_copy(outs_v, outs_hbm.at[pl.ds(cell0 * 16, CELLS_PER_W * 16)])
    pltpu.sync_copy(outg_v, outg_hbm.at[pl.ds(cell0 * 16, CELLS_PER_W * 16)])


def _compose_kernel(l_ref, r_ref, w_ref, b_ref, h_ref):
    full = jnp.concatenate([l_ref[...], r_ref[...]], axis=1)
    h = jnp.dot(full, w_ref[...], preferred_element_type=jnp.float32) + b_ref[...]
    h = jnp.tanh(h)
    norm = jnp.sqrt(jnp.sum(h * h, axis=1, keepdims=True))
    h_ref[...] = h / (norm + 1e-6)


@jax.jit
def kernel(lh, rh, ls, rs, mat, W, b):
    lh_r = lh.reshape(K, BL, N, SIZE)
    rh_r = rh.reshape(K, BL, N, SIZE)
    ls_r = ls.reshape(K, BL, N).transpose(1, 2, 0)
    rs_r = rs.reshape(K, BL, N).transpose(1, 2, 0)

    s_all = pl.pallas_call(
        _score_kernel,
        grid=(BL // CB,),
        in_specs=[
            pl.BlockSpec((K, CB, N, SIZE), lambda i: (0, i, 0, 0)),
            pl.BlockSpec((K, CB, N, SIZE), lambda i: (0, i, 0, 0)),
            pl.BlockSpec((CB, N, K), lambda i: (i, 0, 0)),
            pl.BlockSpec((CB, N, K), lambda i: (i, 0, 0)),
            pl.BlockSpec((SIZE, SIZE), lambda i: (0, 0)),
        ],
        out_specs=pl.BlockSpec((CB * N, K * K), lambda i: (i, 0)),
        out_shape=jax.ShapeDtypeStruct((BL * N, K * K), jnp.float32),
    )(lh_r, rh_r, ls_r, rs_r, mat)

    s_flat = s_all.reshape(BL * N * K * K)
    lh_flat = lh_r.reshape(K * BL * N, SIZE)
    rh_flat = rh_r.reshape(K * BL * N, SIZE)

    sc = functools.partial(
        pl.kernel,
        out_type=(
            jax.ShapeDtypeStruct((BL * K, SIZE), jnp.float32),
            jax.ShapeDtypeStruct((BL * K, SIZE), jnp.float32),
            jax.ShapeDtypeStruct((BL * 16,), jnp.float32),
            jax.ShapeDtypeStruct((BL * 16,), jnp.int32),
        ),
        mesh=plsc.VectorSubcoreMesh(core_axis_name="c", subcore_axis_name="s"),
        scratch_types=[
            pltpu.VMEM((CELLS_PER_W * N * K * K,), jnp.float32),
            pltpu.VMEM((CELLS_PER_W * K,), jnp.int32),
            pltpu.VMEM((CELLS_PER_W * K,), jnp.int32),
            pltpu.VMEM((CELLS_PER_W * K, SIZE), jnp.float32),
            pltpu.VMEM((CELLS_PER_W * K, SIZE), jnp.float32),
            pltpu.VMEM((CELLS_PER_W * 16,), jnp.float32),
            pltpu.VMEM((CELLS_PER_W * 16,), jnp.int32),
            pltpu.SemaphoreType.DMA,
        ],
    )
    lrows, rrows, outs, outg = sc(_sc_select)(s_flat, lh_flat, rh_flat)

    topk_h = pl.pallas_call(
        _compose_kernel,
        in_specs=[
            pl.BlockSpec((BL * K, SIZE), lambda: (0, 0)),
            pl.BlockSpec((BL * K, SIZE), lambda: (0, 0)),
            pl.BlockSpec((2 * SIZE, SIZE), lambda: (0, 0)),
            pl.BlockSpec((1, SIZE), lambda: (0, 0)),
        ],
        out_specs=pl.BlockSpec((BL * K, SIZE), lambda: (0, 0)),
        out_shape=jax.ShapeDtypeStruct((BL * K, SIZE), jnp.float32),
    )(lrows, rrows, W, b.reshape(1, SIZE))

    outs = outs.reshape(BL, 16)
    outg = outg.reshape(BL, 16)
    topk_h = topk_h.reshape(B, L, K, SIZE)
    topk_s = outs[:, :K].reshape(B, L, K, 1)
    n_idx = outg[:, 4:4 + K].reshape(B, L, K)
    lk = outg[:, 8:8 + K].reshape(B, L, K)
    rk = outg[:, 12:12 + K].reshape(B, L, K)
    return (topk_h, topk_s, n_idx, lk, rk)


# speedup vs baseline: 1.0207x; 1.0207x over previous
import functools

import jax
import jax.numpy as jnp
from jax import lax
from jax.experimental import pallas as pl
from jax.experimental.pallas import tpu as pltpu
from jax.experimental.pallas import tpu_sc as plsc

K = 4
B = 8
L = 32
N = 64
SIZE = 128
KN = K * N
BL = B * L
CB = 16
NEG = float("-inf")
BIG = 2 ** 30


def _score_kernel(lh_ref, rh_ref, ls_ref, rs_ref, mat_ref, s_ref):
    mat = mat_ref[...]
    row_i = jax.lax.broadcasted_iota(jnp.int32, (KN, KN), 0)
    col_i = jax.lax.broadcasted_iota(jnp.int32, (KN, KN), 1)
    diag_idx = ((jax.lax.broadcasted_iota(jnp.int32, (KN, 2), 0) & (N - 1))
                + jax.lax.broadcasted_iota(jnp.int32, (KN, 2), 1) * N)
    dn_rowdot = (((1,), (1,)), ((), ()))

    mat_bf = mat.astype(jnp.bfloat16)
    for c in range(CB):
        lh_all = lh_ref[:, c].reshape(KN, SIZE)
        rh_all = rh_ref[:, c].reshape(KN, SIZE)
        lm_all = jnp.dot(lh_all.astype(jnp.bfloat16), mat_bf,
                         preferred_element_type=jnp.float32)
        d_all = jax.lax.dot_general(lm_all.astype(jnp.bfloat16),
                                    rh_all.astype(jnp.bfloat16), dn_rowdot,
                                    preferred_element_type=jnp.float32)
        s4 = jnp.concatenate(
            [jnp.take_along_axis(d_all[:, :2 * N], diag_idx, axis=1),
             jnp.take_along_axis(d_all[:, 2 * N:], diag_idx, axis=1)],
            axis=1)
        sraw = jnp.concatenate([s4[i1 * N:(i1 + 1) * N, :] for i1 in range(K)],
                               axis=1)
        ls4 = ls_ref[c].astype(jnp.bfloat16).astype(jnp.float32)
        rs4 = rs_ref[c].astype(jnp.bfloat16).astype(jnp.float32)
        ls16 = jnp.concatenate(
            [jnp.broadcast_to(ls4[:, i1:i1 + 1], (N, K)) for i1 in range(K)], axis=1)
        rs16 = jnp.concatenate([rs4] * K, axis=1)
        s_ref[c * N:(c + 1) * N, :] = (sraw + ls16) + rs16


CELLS_PER_W = BL // 32
CHUNKS = N


def _sc_select(s_hbm, lh_hbm, rh_hbm,
               lrows_hbm, rrows_hbm, outs_hbm, outg_hbm,
               s_v, lidx_v, ridx_v, lrows_v, rrows_v, outs_v, outg_v, sem):
    info = plsc.get_sparse_core_info()
    nc = info.num_cores
    wid = lax.axis_index("s") * nc + lax.axis_index("c")
    cell0 = wid * CELLS_PER_W

    pltpu.sync_copy(s_hbm.at[pl.ds(cell0 * N * (K * K), CELLS_PER_W * N * K * K)], s_v)

    lane = lax.iota(jnp.int32, 16)
    lvecs = [jnp.zeros((16,), jnp.int32) for _ in range(CELLS_PER_W // 4)]
    rvecs = [jnp.zeros((16,), jnp.int32) for _ in range(CELLS_PER_W // 4)]

    for c in range(CELLS_PER_W):
        sbase = c * (N * K * K)

        def scan_body(i, carry):
            v1, v2, v3, v4, g1, g2, g3, g4 = carry
            x = s_v[pl.ds(sbase + i * 16, 16)]
            fi = i * 16 + lane
            c1 = x > v1
            nv1 = jnp.where(c1, x, v1); r1 = jnp.where(c1, v1, x)
            ng1 = jnp.where(c1, fi, g1); h1 = jnp.where(c1, g1, fi)
            c2 = r1 > v2
            nv2 = jnp.where(c2, r1, v2); r2 = jnp.where(c2, v2, r1)
            ng2 = jnp.where(c2, h1, g2); h2 = jnp.where(c2, g2, h1)
            c3 = r2 > v3
            nv3 = jnp.where(c3, r2, v3); r3 = jnp.where(c3, v3, r2)
            ng3 = jnp.where(c3, h2, g3); h3 = jnp.where(c3, g3, h2)
            c4 = r3 > v4
            nv4 = jnp.where(c4, r3, v4)
            ng4 = jnp.where(c4, h3, g4)
            return nv1, nv2, nv3, nv4, ng1, ng2, ng3, ng4

        neg = jnp.full((16,), NEG)
        big = jnp.full((16,), BIG)
        v1, v2, v3, v4, g1, g2, g3, g4 = lax.fori_loop(
            0, CHUNKS, scan_body, (neg, neg, neg, neg, big, big, big, big))

        res_s = jnp.zeros((16,), jnp.float32)
        res_g = jnp.zeros((16,), jnp.int32)

        dnum = lax.GatherDimensionNumbers(
            offset_dims=(), collapsed_slice_dims=(0,), start_index_map=(0,))

        def perm(x, idx):
            return lax.gather(x, idx[:, None], dnum, (1,),
                              mode=lax.GatherScatterMode.PROMISE_IN_BOUNDS)

        def allmax(x):
            for sh in (1, 2, 4, 8):
                x = jnp.maximum(x, perm(x, lane ^ sh))
            return x

        def allmin(x):
            for sh in (1, 2, 4, 8):
                x = jnp.minimum(x, perm(x, lane ^ sh))
            return x

        for k in range(K):
            m = allmax(v1)
            cand = jnp.where(v1 == m, g1, BIG)
            g = allmin(cand)
            hit = cand == g
            v1 = jnp.where(hit, v2, v1); v2 = jnp.where(hit, v3, v2)
            v3 = jnp.where(hit, v4, v3); v4 = jnp.where(hit, NEG, v4)
            g1 = jnp.where(hit, g2, g1); g2 = jnp.where(hit, g3, g2)
            g3 = jnp.where(hit, g4, g3); g4 = jnp.where(hit, BIG, g4)
            n_ = g >> 4
            lk = (g >> 2) & 3
            rk = g & 3
            res_s = jnp.where(lane == k, m, res_s)
            res_g = jnp.where(lane == k, g, res_g)
            res_g = jnp.where(lane == 4 + k, n_, res_g)
            res_g = jnp.where(lane == 8 + k, lk, res_g)
            res_g = jnp.where(lane == 12 + k, rk, res_g)
            lrow = (lk * BL + (cell0 + c)) * N + n_
            rrow = (rk * BL + (cell0 + c)) * N + n_
            slot = (c % 4) * 4 + k
            lvecs[c // 4] = jnp.where(lane == slot, lrow, lvecs[c // 4])
            rvecs[c // 4] = jnp.where(lane == slot, rrow, rvecs[c // 4])

        outs_v[pl.ds(c * 16, 16)] = res_s
        outg_v[pl.ds(c * 16, 16)] = res_g

    for q in range(CELLS_PER_W // 4):
        lidx_v[pl.ds(q * 16, 16)] = lvecs[q]
        ridx_v[pl.ds(q * 16, 16)] = rvecs[q]

    pltpu.async_copy(lh_hbm.at[lidx_v], lrows_v, sem).wait()
    pltpu.async_copy(rh_hbm.at[ridx_v], rrows_v, sem).wait()

    rows0 = wid * (CELLS_PER_W * K)
    pltpu.sync_copy(lrows_v, lrows_hbm.at[pl.ds(rows0, CELLS_PER_W * K)])
    pltpu.sync_copy(rrows_v, rrows_hbm.at[pl.ds(rows0, CELLS_PER_W * K)])
    pltpu.sync_copy(outs_v, outs_hbm.at[pl.ds(cell0 * 16, CELLS_PER_W * 16)])
    pltpu.sync_copy(outg_v, outg_hbm.at[pl.ds(cell0 * 16, CELLS_PER_W * 16)])


def _compose_kernel(l_ref, r_ref, w_ref, b_ref, h_ref):
    full = jnp.concatenate([l_ref[...], r_ref[...]], axis=1)
    h = jnp.dot(full, w_ref[...], preferred_element_type=jnp.float32) + b_ref[...]
    h = jnp.tanh(h)
    norm = jnp.sqrt(jnp.sum(h * h, axis=1, keepdims=True))
    h_ref[...] = h / (norm + 1e-6)


@jax.jit
def kernel(lh, rh, ls, rs, mat, W, b):
    lh_r = lh.reshape(K, BL, N, SIZE)
    rh_r = rh.reshape(K, BL, N, SIZE)
    ls_r = ls.reshape(K, BL, N).transpose(1, 2, 0)
    rs_r = rs.reshape(K, BL, N).transpose(1, 2, 0)

    s_all = pl.pallas_call(
        _score_kernel,
        grid=(BL // CB,),
        in_specs=[
            pl.BlockSpec((K, CB, N, SIZE), lambda i: (0, i, 0, 0)),
            pl.BlockSpec((K, CB, N, SIZE), lambda i: (0, i, 0, 0)),
            pl.BlockSpec((CB, N, K), lambda i: (i, 0, 0)),
            pl.BlockSpec((CB, N, K), lambda i: (i, 0, 0)),
            pl.BlockSpec((SIZE, SIZE), lambda i: (0, 0)),
        ],
        out_specs=pl.BlockSpec((CB * N, K * K), lambda i: (i, 0)),
        out_shape=jax.ShapeDtypeStruct((BL * N, K * K), jnp.float32),
    )(lh_r, rh_r, ls_r, rs_r, mat)

    s_flat = s_all.reshape(BL * N * K * K)
    lh_flat = lh_r.reshape(K * BL * N, SIZE)
    rh_flat = rh_r.reshape(K * BL * N, SIZE)

    sc = functools.partial(
        pl.kernel,
        out_type=(
            jax.ShapeDtypeStruct((BL * K, SIZE), jnp.float32),
            jax.ShapeDtypeStruct((BL * K, SIZE), jnp.float32),
            jax.ShapeDtypeStruct((BL * 16,), jnp.float32),
            jax.ShapeDtypeStruct((BL * 16,), jnp.int32),
        ),
        mesh=plsc.VectorSubcoreMesh(core_axis_name="c", subcore_axis_name="s"),
        scratch_types=[
            pltpu.VMEM((CELLS_PER_W * N * K * K,), jnp.float32),
            pltpu.VMEM((CELLS_PER_W * K,), jnp.int32),
            pltpu.VMEM((CELLS_PER_W * K,), jnp.int32),
            pltpu.VMEM((CELLS_PER_W * K, SIZE), jnp.float32),
            pltpu.VMEM((CELLS_PER_W * K, SIZE), jnp.float32),
            pltpu.VMEM((CELLS_PER_W * 16,), jnp.float32),
            pltpu.VMEM((CELLS_PER_W * 16,), jnp.int32),
            pltpu.SemaphoreType.DMA,
        ],
    )
    lrows, rrows, outs, outg = sc(_sc_select)(s_flat, lh_flat, rh_flat)

    topk_h = pl.pallas_call(
        _compose_kernel,
        in_specs=[
            pl.BlockSpec((BL * K, SIZE), lambda: (0, 0)),
            pl.BlockSpec((BL * K, SIZE), lambda: (0, 0)),
            pl.BlockSpec((2 * SIZE, SIZE), lambda: (0, 0)),
            pl.BlockSpec((1, SIZE), lambda: (0, 0)),
        ],
        out_specs=pl.BlockSpec((BL * K, SIZE), lambda: (0, 0)),
        out_shape=jax.ShapeDtypeStruct((BL * K, SIZE), jnp.float32),
    )(lrows, rrows, W, b.reshape(1, SIZE))

    outs = outs.reshape(BL, 16)
    outg = outg.reshape(BL, 16)
    topk_h = topk_h.reshape(B, L, K, SIZE)
    topk_s = outs[:, :K].reshape(B, L, K, 1)
    n_idx = outg[:, 4:4 + K].reshape(B, L, K)
    lk = outg[:, 8:8 + K].reshape(B, L, K)
    rk = outg[:, 12:12 + K].reshape(B, L, K)
    return (topk_h, topk_s, n_idx, lk, rk)
